# baseline (speedup 1.0000x reference)
import functools

import jax
import jax.numpy as jnp
from jax import lax
from jax.experimental import pallas as pl
from jax.experimental.pallas import tpu as pltpu
from jax.experimental.pallas import tpu_sc as plsc

B, N, M, C = 4096, 200, 50, 128
L = 16
NW = 32
NSPLIT = 2
NB = B // NSPLIT
BPW = NB // NW
CHUNKS = (M * C) // L


def _body(base, x_hbm, idx_hbm, out_hbm,
          xv0, xv1, iv0, iv1, ov0, ov1,
          sx0, sx1, si0, si1, so0, so1):
    wid = lax.axis_index("s") * 2 + lax.axis_index("c")
    lb = wid * BPW
    lane = lax.broadcasted_iota(jnp.int32, (L,), 0)
    xv, iv, ov = (xv0, xv1), (iv0, iv1), (ov0, ov1)
    sx, si, so = (sx0, sx1), (si0, si1), (so0, so1)

    for p in range(2):
        pltpu.async_copy(x_hbm.at[base + lb + p], xv[p], sx[p])
        pltpu.async_copy(idx_hbm.at[lb + p], iv[p], si[p])

    def gather_batch(src_x, src_i, dst_o):
        @plsc.parallel_loop(0, CHUNKS, unroll=8)
        def _(j):
            row = j // (C // L)
            cs = (j % (C // L)) * L
            idxv = src_i[row, pl.ds(cs, L)]
            dst_o[row, pl.ds(cs, L)] = plsc.load_gather(
                src_x, [idxv, cs + lane])

    def step(t, carry):
        for p in range(2):
            b = lb + 2 * t + p
            pltpu.make_async_copy(x_hbm.at[base + b], xv[p], sx[p]).wait()
            pltpu.make_async_copy(idx_hbm.at[b], iv[p], si[p]).wait()

            @pl.when(t > 0)
            def _():
                pltpu.make_async_copy(ov[p], out_hbm.at[b - 2], so[p]).wait()

            gather_batch(xv[p], iv[p], ov[p])
            pltpu.async_copy(ov[p], out_hbm.at[b], so[p])

            @pl.when(t < BPW // 2 - 1)
            def _():
                pltpu.async_copy(x_hbm.at[base + b + 2], xv[p], sx[p])
                pltpu.async_copy(idx_hbm.at[b + 2], iv[p], si[p])
        return carry

    lax.fori_loop(0, BPW // 2, step, 0)
    for p in range(2):
        pltpu.make_async_copy(ov[p], out_hbm.at[lb + BPW - 2 + p],
                              so[p]).wait()


def _make_call(base):
    mesh = plsc.VectorSubcoreMesh(core_axis_name="c", subcore_axis_name="s")
    return functools.partial(
        pl.kernel,
        out_type=jax.ShapeDtypeStruct((NB, M, C), jnp.float32),
        mesh=mesh,
        scratch_types=[
            pltpu.VMEM((N, C), jnp.float32),
            pltpu.VMEM((N, C), jnp.float32),
            pltpu.VMEM((M, C), jnp.int32),
            pltpu.VMEM((M, C), jnp.int32),
            pltpu.VMEM((M, C), jnp.float32),
            pltpu.VMEM((M, C), jnp.float32),
            pltpu.SemaphoreType.DMA,
            pltpu.SemaphoreType.DMA,
            pltpu.SemaphoreType.DMA,
            pltpu.SemaphoreType.DMA,
            pltpu.SemaphoreType.DMA,
            pltpu.SemaphoreType.DMA,
        ],
        compiler_params=pltpu.CompilerParams(needs_layout_passes=False),
    )(functools.partial(_body, base))


@jax.jit
def _gather_sc(x, idx):
    outs = []
    for s in range(NSPLIT):
        f = _make_call(s * NB)
        outs.append(f(x, idx[s * NB:(s + 1) * NB]))
    return jnp.concatenate(outs, axis=0)


def kernel(x, indices):
    return _gather_sc(x, indices.astype(jnp.int32))


# speedup vs baseline: 1.4207x; 1.4207x over previous
import functools

import jax
import jax.numpy as jnp
from jax import lax
from jax.experimental import pallas as pl
from jax.experimental.pallas import tpu as pltpu
from jax.experimental.pallas import tpu_sc as plsc

B, N, M, C = 4096, 200, 50, 128
MP = 56
L = 16
NW = 32
BPW = B // NW
CHUNKS = (M * C) // L


def _body(x_hbm, idx_hbm, out_hbm,
          xv0, xv1, iv0, iv1, ov0, ov1,
          sx0, sx1, si0, si1, so0, so1):
    wid = lax.axis_index("s") * 2 + lax.axis_index("c")
    base_b = wid * BPW
    lane = lax.broadcasted_iota(jnp.int32, (L,), 0)
    xv, iv, ov = (xv0, xv1), (iv0, iv1), (ov0, ov1)
    sx, si, so = (sx0, sx1), (si0, si1), (so0, so1)

    for p in range(2):
        pltpu.async_copy(x_hbm.at[base_b + p], xv[p], sx[p])
        pltpu.async_copy(idx_hbm.at[base_b + p], iv[p], si[p])

    def gather_batch(src_x, src_i, dst_o):
        @plsc.parallel_loop(0, CHUNKS, unroll=8)
        def _(j):
            row = j // (C // L)
            cs = (j % (C // L)) * L
            idxv = src_i[row, pl.ds(cs, L)]
            dst_o[row, pl.ds(cs, L)] = plsc.load_gather(
                src_x, [idxv, cs + lane])

    def step(t, carry):
        for p in range(2):
            b = base_b + 2 * t + p
            pltpu.make_async_copy(x_hbm.at[b], xv[p], sx[p]).wait()
            pltpu.make_async_copy(idx_hbm.at[b], iv[p], si[p]).wait()

            @pl.when(t > 0)
            def _():
                pltpu.make_async_copy(ov[p], out_hbm.at[b - 2], so[p]).wait()

            gather_batch(xv[p], iv[p], ov[p])
            pltpu.async_copy(ov[p], out_hbm.at[b], so[p])

            @pl.when(t < BPW // 2 - 1)
            def _():
                pltpu.async_copy(x_hbm.at[b + 2], xv[p], sx[p])
                pltpu.async_copy(idx_hbm.at[b + 2], iv[p], si[p])
        return carry

    lax.fori_loop(0, BPW // 2, step, 0)
    for p in range(2):
        pltpu.make_async_copy(ov[p], out_hbm.at[base_b + BPW - 2 + p],
                              so[p]).wait()


@jax.jit
def _gather_sc(x, idx):
    mesh = plsc.VectorSubcoreMesh(core_axis_name="c", subcore_axis_name="s")
    f = functools.partial(
        pl.kernel,
        out_type=jax.ShapeDtypeStruct((B, MP, C), jnp.float32),
        mesh=mesh,
        scratch_types=[
            pltpu.VMEM((N, C), jnp.float32),
            pltpu.VMEM((N, C), jnp.float32),
            pltpu.VMEM((M, C), jnp.int32),
            pltpu.VMEM((M, C), jnp.int32),
            pltpu.VMEM((MP, C), jnp.float32),
            pltpu.VMEM((MP, C), jnp.float32),
            pltpu.SemaphoreType.DMA,
            pltpu.SemaphoreType.DMA,
            pltpu.SemaphoreType.DMA,
            pltpu.SemaphoreType.DMA,
            pltpu.SemaphoreType.DMA,
            pltpu.SemaphoreType.DMA,
        ],
        compiler_params=pltpu.CompilerParams(needs_layout_passes=False),
    )(_body)
    return f(x, idx)[:, :M, :]


def kernel(x, indices):
    return _gather_sc(x, indices.astype(jnp.int32))


# speedup vs baseline: 1.5119x; 1.0642x over previous
import functools

import jax
import jax.numpy as jnp
from jax import lax
from jax.experimental import pallas as pl
from jax.experimental.pallas import tpu as pltpu
from jax.experimental.pallas import tpu_sc as plsc

B, N, M, C = 4096, 200, 50, 128
L = 16
NW = 32
BPW = B // NW
CHUNKS = (M * C) // L


def _body(x_hbm, idx_hbm, out_hbm,
          xv0, xv1, iv0, iv1, ov0, ov1,
          sx0, sx1, si0, si1, so0, so1):
    wid = lax.axis_index("s") * 2 + lax.axis_index("c")
    base_b = wid * BPW
    lane = lax.broadcasted_iota(jnp.int32, (L,), 0)
    xv, iv, ov = (xv0, xv1), (iv0, iv1), (ov0, ov1)
    sx, si, so = (sx0, sx1), (si0, si1), (so0, so1)

    for p in range(2):
        pltpu.async_copy(x_hbm.at[base_b + p], xv[p], sx[p])
        pltpu.async_copy(idx_hbm.at[base_b + p], iv[p], si[p])

    def gather_batch(src_x, src_i, dst_o):
        @plsc.parallel_loop(0, CHUNKS, unroll=8)
        def _(j):
            row = j // (C // L)
            cs = (j % (C // L)) * L
            idxv = src_i[row, pl.ds(cs, L)]
            dst_o[row, pl.ds(cs, L)] = plsc.load_gather(
                src_x, [idxv, cs + lane])

    def step(t, carry):
        for p in range(2):
            b = base_b + 2 * t + p
            pltpu.make_async_copy(x_hbm.at[b], xv[p], sx[p]).wait()
            pltpu.make_async_copy(idx_hbm.at[b], iv[p], si[p]).wait()

            @pl.when(t > 0)
            def _():
                pltpu.make_async_copy(ov[p], out_hbm.at[b - 2], so[p]).wait()

            gather_batch(xv[p], iv[p], ov[p])
            pltpu.async_copy(ov[p], out_hbm.at[b], so[p])

            @pl.when(t < BPW // 2 - 1)
            def _():
                pltpu.async_copy(x_hbm.at[b + 2], xv[p], sx[p])
                pltpu.async_copy(idx_hbm.at[b + 2], iv[p], si[p])
        return carry

    lax.fori_loop(0, BPW // 2, step, 0)
    for p in range(2):
        pltpu.make_async_copy(ov[p], out_hbm.at[base_b + BPW - 2 + p],
                              so[p]).wait()


@jax.jit
def _gather_sc(x, idx):
    mesh = plsc.VectorSubcoreMesh(core_axis_name="c", subcore_axis_name="s")
    f = functools.partial(
        pl.kernel,
        out_type=jax.ShapeDtypeStruct((B, M, C), jnp.float32),
        mesh=mesh,
        scratch_types=[
            pltpu.VMEM((N, C), jnp.float32),
            pltpu.VMEM((N, C), jnp.float32),
            pltpu.VMEM((M, C), jnp.int32),
            pltpu.VMEM((M, C), jnp.int32),
            pltpu.VMEM((M, C), jnp.float32),
            pltpu.VMEM((M, C), jnp.float32),
            pltpu.SemaphoreType.DMA,
            pltpu.SemaphoreType.DMA,
            pltpu.SemaphoreType.DMA,
            pltpu.SemaphoreType.DMA,
            pltpu.SemaphoreType.DMA,
            pltpu.SemaphoreType.DMA,
        ],
        compiler_params=pltpu.CompilerParams(needs_layout_passes=False),
    )(_body)
    return f(x, idx)


def kernel(x, indices):
    return _gather_sc(x, indices.astype(jnp.int32))
